# baseline (speedup 1.0000x reference)
import functools

import jax
import jax.numpy as jnp
from jax.experimental import pallas as pl
from jax.experimental.pallas import tpu as pltpu

_K = 1024
_DIM = 50257
_Q = 32
_D_BLK = 2048


def _knn_body(q_ref, a_ref, lab_ref, out_ref, ent_acc, cross_acc):
    j = pl.program_id(0)
    nd = pl.num_programs(0)

    @pl.when(j == 0)
    def _init():
        ent_acc[...] = jnp.zeros_like(ent_acc)
        cross_acc[...] = jnp.zeros_like(cross_acc)

    a = a_ref[...]
    q = q_ref[...]
    col = j * _D_BLK + jax.lax.broadcasted_iota(jnp.int32, (1, _D_BLK), 1)
    mask = col < _DIM
    a_m = jnp.where(mask, a, 1.0)
    lq = jnp.where(mask, jnp.log(q), 0.0)
    ent_acc[...] += jnp.sum(a_m * jnp.log(a_m), axis=1)
    cross_acc[...] += jax.lax.dot_general(
        a_m, lq, (((1,), (1,)), ((), ())),
        preferred_element_type=jnp.float32)

    @pl.when(j == nd - 1)
    def _finish():
        ent = ent_acc[...] / _DIM
        cross = cross_acc[...] / _DIM
        kl = ent[:, None] - cross
        m = jnp.min(kl, axis=0)
        row = jax.lax.broadcasted_iota(jnp.int32, (_K, _Q), 0)
        idx = jnp.min(jnp.where(kl == m[None, :], row, _K), axis=0)
        lab = lab_ref[...]
        out_ref[...] = jnp.sum(
            jnp.where(row == idx[None, :], lab, 0), axis=0)


@jax.jit
def kernel(query, queue_anchor, queue_label):
    nd = (_DIM + _D_BLK - 1) // _D_BLK
    lab2 = queue_label.reshape(_K, 1)
    return pl.pallas_call(
        _knn_body,
        grid=(nd,),
        in_specs=[
            pl.BlockSpec((_Q, _D_BLK), lambda j: (0, j)),
            pl.BlockSpec((_K, _D_BLK), lambda j: (0, j)),
            pl.BlockSpec((_K, 1), lambda j: (0, 0)),
        ],
        out_specs=pl.BlockSpec((_Q,), lambda j: (0,)),
        out_shape=jax.ShapeDtypeStruct((_Q,), jnp.int32),
        scratch_shapes=[
            pltpu.VMEM((_K,), jnp.float32),
            pltpu.VMEM((_K, _Q), jnp.float32),
        ],
        compiler_params=pltpu.CompilerParams(
            dimension_semantics=("arbitrary",)),
    )(query, queue_anchor, lab2)


# speedup vs baseline: 1.0155x; 1.0155x over previous
import functools

import jax
import jax.numpy as jnp
from jax.experimental import pallas as pl
from jax.experimental.pallas import tpu as pltpu

_K = 1024
_DIM = 50257
_Q = 32
_D_BLK = 4096


def _knn_body(q_ref, a_ref, lab_ref, out_ref, ent_acc, cross_acc):
    j = pl.program_id(0)
    nd = pl.num_programs(0)

    @pl.when(j == 0)
    def _init():
        ent_acc[...] = jnp.zeros_like(ent_acc)
        cross_acc[...] = jnp.zeros_like(cross_acc)

    a = a_ref[...]
    q = q_ref[...]
    col = j * _D_BLK + jax.lax.broadcasted_iota(jnp.int32, (1, _D_BLK), 1)
    mask = col < _DIM
    a_m = jnp.where(mask, a, 1.0)
    lq = jnp.where(mask, jnp.log(q), 0.0)
    ent_acc[...] += jnp.sum(a_m * jnp.log(a_m), axis=1)
    cross_acc[...] += jax.lax.dot_general(
        a_m, lq, (((1,), (1,)), ((), ())),
        preferred_element_type=jnp.float32)

    @pl.when(j == nd - 1)
    def _finish():
        ent = ent_acc[...] / _DIM
        cross = cross_acc[...] / _DIM
        kl = ent[:, None] - cross
        m = jnp.min(kl, axis=0)
        row = jax.lax.broadcasted_iota(jnp.int32, (_K, _Q), 0)
        idx = jnp.min(jnp.where(kl == m[None, :], row, _K), axis=0)
        lab = lab_ref[...]
        out_ref[...] = jnp.sum(
            jnp.where(row == idx[None, :], lab, 0), axis=0)


@jax.jit
def kernel(query, queue_anchor, queue_label):
    nd = (_DIM + _D_BLK - 1) // _D_BLK
    lab2 = queue_label.reshape(_K, 1)
    return pl.pallas_call(
        _knn_body,
        grid=(nd,),
        in_specs=[
            pl.BlockSpec((_Q, _D_BLK), lambda j: (0, j)),
            pl.BlockSpec((_K, _D_BLK), lambda j: (0, j)),
            pl.BlockSpec((_K, 1), lambda j: (0, 0)),
        ],
        out_specs=pl.BlockSpec((_Q,), lambda j: (0,)),
        out_shape=jax.ShapeDtypeStruct((_Q,), jnp.int32),
        scratch_shapes=[
            pltpu.VMEM((_K,), jnp.float32),
            pltpu.VMEM((_K, _Q), jnp.float32),
        ],
        compiler_params=pltpu.CompilerParams(
            dimension_semantics=("arbitrary",)),
    )(query, queue_anchor, lab2)
